# baseline (speedup 1.0000x reference)
import functools

import jax
import jax.numpy as jnp
from jax import lax
from jax.experimental import pallas as pl
from jax.experimental.pallas import tpu as pltpu
from jax.experimental.pallas import tpu_sc as plsc

_VOCAB = 1000000
_EMB = 64
_B = 4096
_L = 200
_SCALE = 8.0

_NC = 2
_NS = 16
_NW = _NC * _NS
_BPW = _B // _NW
_DEPTH = 4


def _emb_body(tokens_hbm, table_hbm, out_hbm, tok_v, idxt, gbuf, obuf, *sems):
    gsems = sems[:_DEPTH]
    osems = sems[_DEPTH:]

    wid = lax.axis_index("s") * _NC + lax.axis_index("c")

    pltpu.sync_copy(tokens_hbm.at[pl.ds(wid * _BPW, _BPW)], tok_v)

    iota = lax.iota(jnp.int32, 16)

    def tr_l(l, c):
        col = jnp.full((16,), l, jnp.int32)
        for b16 in range(_BPW // 16):
            v = plsc.load_gather(tok_v, [iota + b16 * 16, col])
            idxt[l, pl.ds(b16 * 16, 16)] = v
        return c

    lax.fori_loop(0, _L, tr_l, 0)

    def start_gather(l, k):
        pltpu.async_copy(table_hbm.at[idxt.at[l]], gbuf.at[k], gsems[k])

    def wait_gather(l, k):
        pltpu.make_async_copy(
            table_hbm.at[idxt.at[l]], gbuf.at[k], gsems[k]
        ).wait()

    def start_out(l, k):
        for cr in range(_EMB // 8):
            pltpu.async_copy(
                obuf.at[k, pl.ds(cr * 8, 8)],
                out_hbm.at[l, cr, wid],
                osems[k],
            )

    def wait_out(l, k):
        for cr in range(_EMB // 8):
            pltpu.make_async_copy(
                obuf.at[k, pl.ds(cr * 8, 8)],
                out_hbm.at[l, cr, wid],
                osems[k],
            ).wait()

    for k in range(_DEPTH):
        start_gather(k, k)

    def round_body(i, carry):
        for k in range(_DEPTH):
            l = _DEPTH * i + k
            wait_gather(l, k)

            @pl.when(l >= _DEPTH)
            def _():
                wait_out(l - _DEPTH, k)

            def chan(c, cc):
                col = jnp.full((16,), c, jnp.int32)
                for b16 in range(_BPW // 16):
                    v = plsc.load_gather(gbuf.at[k], [iota + b16 * 16, col])
                    obuf[k, c, pl.ds(b16 * 16, 16)] = v * _SCALE
                return cc

            lax.fori_loop(0, _EMB, chan, 0)

            start_out(l, k)

            @pl.when(l + _DEPTH < _L)
            def _():
                start_gather(l + _DEPTH, k)

        return carry

    lax.fori_loop(0, _L // _DEPTH, round_body, 0)

    for k in range(_DEPTH):
        wait_out(_L - _DEPTH + k, k)


@jax.jit
def _embed(tokens, table):
    run = functools.partial(
        pl.kernel,
        mesh=plsc.VectorSubcoreMesh(core_axis_name="c", subcore_axis_name="s"),
        out_type=jax.ShapeDtypeStruct((_L, _EMB // 8, _NW, 8, 128), jnp.float32),
        scratch_types=[
            pltpu.VMEM((_BPW, _L), jnp.int32),
            pltpu.VMEM((_L, _BPW), jnp.int32),
            pltpu.VMEM((_DEPTH, _BPW, _EMB), jnp.float32),
            pltpu.VMEM((_DEPTH, _EMB, _BPW), jnp.float32),
        ]
        + [pltpu.SemaphoreType.DMA] * (2 * _DEPTH),
        compiler_params=pltpu.CompilerParams(
            use_tc_tiling_on_sc=False, needs_layout_passes=False
        ),
    )(_emb_body)
    return run(tokens, table)


def kernel(tokens, table):
    th = _embed(tokens, table)
    return th.transpose((2, 4, 0, 1, 3)).reshape(_B, _L, _EMB)
